# baseline (speedup 1.0000x reference)
import functools

import jax
import jax.numpy as jnp
from jax import lax
from jax.experimental import pallas as pl
from jax.experimental.pallas import tpu as pltpu
from jax.experimental.pallas import tpu_sc as plsc

VOCAB = 1000000
D = 64
PAD = 0
BATCH = 4096
SEQ = 200
NW = 32
BB = BATCH // NW


def _emb_body(x_hbm, table_hbm, out_hbm, idx_v, rows_v, rows_t, sg0, sg1,
              ss0, ss1):
    w = lax.axis_index("s") * 2 + lax.axis_index("c")
    pltpu.sync_copy(x_hbm.at[w], idx_v)

    def gather(g, p, sem):
        pltpu.async_copy(table_hbm.at[idx_v.at[g]], rows_v.at[p], sem)

    def gather_wait(p, sem):
        pltpu.make_async_copy(table_hbm.at[pl.ds(0, BB)], rows_v.at[p],
                              sem).wait()

    def scatter(g, p, sem):
        for t in range(8):
            pltpu.async_copy(rows_t.at[p, t], out_hbm.at[g, t, w], sem)

    def scatter_wait(p, sem):
        for t in range(8):
            pltpu.make_async_copy(out_hbm.at[0, t, w], rows_t.at[p, t],
                                  sem).wait()

    def transpose(g, p):
        lanes = lax.iota(jnp.int32, 16)
        z = jnp.zeros((16,), jnp.float32)

        def tbody(h, c):
            iv = idx_v[g, pl.ds(h * 16, 16)]
            m = iv == PAD
            npad = plsc.all_reduce_population_count(m)

            @pl.when(lax.squeeze(lax.slice(npad, (0,), (1,)), (0,)) > 0)
            def _():
                r16 = h * 16 + lanes
                for j in range(D):
                    plsc.store_scatter(
                        rows_v.at[p], [r16, jnp.full((16,), j, jnp.int32)],
                        z, mask=m)

            for u in range(16):
                bi = h * 16 + u
                bv = jax.lax.broadcast(bi, (16,))
                for dc in range(4):
                    v = rows_v[p, bi, pl.ds(dc * 16, 16)]
                    dd = dc * 16 + lanes
                    plsc.store_scatter(rows_t.at[p], [dd >> 3, dd & 7, bv], v)
            return c

        lax.fori_loop(0, BB // 16, tbody, 0)

    gather(0, 0, sg0)
    gather(1, 1, sg1)

    def pair(i, carry):
        for p, sg, ss in ((0, sg0, ss0), (1, sg1, ss1)):
            g = i * 2 + p
            gather_wait(p, sg)

            @pl.when(i > 0)
            def _():
                scatter_wait(p, ss)

            transpose(g, p)

            @pl.when(i < SEQ // 2 - 1)
            def _():
                gather(g + 2, p, sg)

            scatter(g, p, ss)
        return carry

    lax.fori_loop(0, SEQ // 2, pair, 0)
    scatter_wait(0, ss0)
    scatter_wait(1, ss1)


@jax.jit
def kernel(x, table):
    xi = x.astype(jnp.int32).reshape(NW, BB, SEQ).transpose(0, 2, 1)
    k = functools.partial(
        pl.kernel,
        mesh=plsc.VectorSubcoreMesh(core_axis_name="c", subcore_axis_name="s"),
        out_type=jax.ShapeDtypeStruct((SEQ, 8, NW, 8, BB), jnp.float32),
        scratch_types=[
            pltpu.VMEM((SEQ, BB), jnp.int32),
            pltpu.VMEM((2, BB, D), jnp.float32),
            pltpu.VMEM((2, 8, 8, BB), jnp.float32),
            pltpu.SemaphoreType.DMA,
            pltpu.SemaphoreType.DMA,
            pltpu.SemaphoreType.DMA,
            pltpu.SemaphoreType.DMA,
        ],
        compiler_params=pltpu.CompilerParams(
            needs_layout_passes=False, use_tc_tiling_on_sc=False),
    )(_emb_body)
    out5 = k(xi, table)
    return out5.transpose(2, 4, 0, 1, 3).reshape(BATCH, SEQ, D)
